# baseline (speedup 1.0000x reference)
import functools

import jax
import jax.numpy as jnp
from jax.experimental import pallas as pl
from jax.experimental.pallas import tpu as pltpu

MAX_K = 128
NEG_INF = -1e30
_SEG = 128
_LANE = 1024
_VP = _SEG * _LANE


def _topk_kernel(x_ref, tc_ref, pen_ref, temp_ref, vals_out, idx_out,
                 y_ref, m_ref, vals_s, idx_s, *, valid_v):
    b = pl.program_id(0)
    x = x_ref[0]
    tc = tc_ref[0]
    pen = pen_ref[b, 0]
    inv_t = 1.0 / temp_ref[b, 0]

    seg_iota = jax.lax.broadcasted_iota(jnp.int32, (_SEG, _LANE), 0)
    lane_iota2 = jax.lax.broadcasted_iota(jnp.int32, (_SEG, _LANE), 1)
    vocab_pos = seg_iota * _LANE + lane_iota2

    penalized = jnp.where(x > 0, x * (1.0 / pen), x * pen)
    x1 = jnp.where(tc > 0, penalized, x) * inv_t
    x1 = jnp.where(vocab_pos < valid_v, x1, -jnp.inf)

    y_ref[...] = x1
    m_ref[...] = jnp.max(x1, axis=1, keepdims=True)

    seg_ids = jax.lax.broadcasted_iota(jnp.int32, (_SEG, 1), 0)
    lane_ids = jax.lax.broadcasted_iota(jnp.int32, (1, _LANE), 1)

    def body(k, carry):
        m = m_ref[...]
        g = jnp.max(m)
        s = jnp.min(jnp.where(m == g, seg_ids, _SEG))
        row = y_ref[pl.ds(s, 1), :]
        l = jnp.min(jnp.where(row == g, lane_ids, _LANE))
        vals_s[pl.ds(k, 1), :] = jnp.full((1, 1), g)
        idx_s[pl.ds(k, 1), :] = jnp.full((1, 1), s * _LANE + l,
                                         dtype=jnp.int32)
        newrow = jnp.where(lane_ids == l, -jnp.inf, row)
        y_ref[pl.ds(s, 1), :] = newrow
        m_ref[pl.ds(s, 1), :] = jnp.max(newrow, axis=1, keepdims=True)
        return carry

    jax.lax.fori_loop(0, MAX_K, body, 0)
    vals_out[0] = vals_s[...]
    idx_out[0] = idx_s[...]


def _sample_kernel(vals_ref, idx_ref, topk_ref, topp_ref, rnd_ref, gidx_ref,
                   out_ref):
    vals = vals_ref[...]
    idxs = idx_ref[...]
    B = vals.shape[0]

    pos = jax.lax.broadcasted_iota(jnp.int32, (B, MAX_K), 1)
    tl = jnp.where(pos < topk_ref[...], vals, NEG_INF)
    m = jnp.max(tl, axis=1, keepdims=True)
    e = jnp.exp(tl - m)
    probs = e / jnp.sum(e, axis=1, keepdims=True)

    ti = jax.lax.broadcasted_iota(jnp.int32, (MAX_K, MAX_K), 0)
    tj = jax.lax.broadcasted_iota(jnp.int32, (MAX_K, MAX_K), 1)
    tri = (ti <= tj).astype(jnp.float32)

    cum = jax.lax.dot(probs, tri, precision=jax.lax.Precision.HIGHEST)
    keep = (cum - probs) < topp_ref[...]
    p2 = jnp.where(keep, probs, 0.0)
    p2 = p2 / jnp.sum(p2, axis=1, keepdims=True)

    s_iota = jax.lax.broadcasted_iota(jnp.int32, rnd_ref.shape, 1)
    r = jnp.sum(jnp.where(s_iota == gidx_ref[...], rnd_ref[...], 0.0),
                axis=1, keepdims=True)

    cdf = jax.lax.dot(p2, tri, precision=jax.lax.Precision.HIGHEST)
    sel = jnp.sum((cdf < r).astype(jnp.int32), axis=1, keepdims=True)
    sel = jnp.clip(sel, 0, MAX_K - 1)
    out_ref[...] = jnp.sum(jnp.where(pos == sel, idxs, 0),
                           axis=1, keepdims=True)


@jax.jit
def kernel(logits, generated_index, temperature, top_k, top_p,
           pre_generated_randoms, repetition_penalty, token_count):
    B, _, V = logits.shape
    S = pre_generated_randoms.shape[1]
    lg = jnp.squeeze(logits, axis=1).astype(jnp.float32)
    pad = _VP - V
    lg_p = jnp.pad(lg, ((0, 0), (0, pad))).reshape(B, _SEG, _LANE)
    tc_p = jnp.pad(token_count.astype(jnp.int32),
                   ((0, 0), (0, pad))).reshape(B, _SEG, _LANE)

    vals3, idx3 = pl.pallas_call(
        functools.partial(_topk_kernel, valid_v=V),
        grid=(B,),
        in_specs=[
            pl.BlockSpec((1, _SEG, _LANE), lambda b: (b, 0, 0)),
            pl.BlockSpec((1, _SEG, _LANE), lambda b: (b, 0, 0)),
            pl.BlockSpec(memory_space=pltpu.SMEM),
            pl.BlockSpec(memory_space=pltpu.SMEM),
        ],
        out_specs=[
            pl.BlockSpec((1, MAX_K, 1), lambda b: (b, 0, 0)),
            pl.BlockSpec((1, MAX_K, 1), lambda b: (b, 0, 0)),
        ],
        out_shape=[
            jax.ShapeDtypeStruct((B, MAX_K, 1), jnp.float32),
            jax.ShapeDtypeStruct((B, MAX_K, 1), jnp.int32),
        ],
        scratch_shapes=[
            pltpu.VMEM((_SEG, _LANE), jnp.float32),
            pltpu.VMEM((_SEG, 1), jnp.float32),
            pltpu.VMEM((MAX_K, 1), jnp.float32),
            pltpu.VMEM((MAX_K, 1), jnp.int32),
        ],
    )(lg_p, tc_p, repetition_penalty.astype(jnp.float32),
      temperature.astype(jnp.float32))

    vals = vals3.reshape(B, MAX_K)
    idxs = idx3.reshape(B, MAX_K)

    sampled = pl.pallas_call(
        _sample_kernel,
        in_specs=[
            pl.BlockSpec((B, MAX_K), lambda: (0, 0)),
            pl.BlockSpec((B, MAX_K), lambda: (0, 0)),
            pl.BlockSpec((B, 1), lambda: (0, 0)),
            pl.BlockSpec((B, 1), lambda: (0, 0)),
            pl.BlockSpec((B, S), lambda: (0, 0)),
            pl.BlockSpec((B, 1), lambda: (0, 0)),
        ],
        out_specs=pl.BlockSpec((B, 1), lambda: (0, 0)),
        out_shape=jax.ShapeDtypeStruct((B, 1), jnp.int32),
    )(vals, idxs, top_k.astype(jnp.int32), top_p.astype(jnp.float32),
      pre_generated_randoms.astype(jnp.float32),
      generated_index.astype(jnp.int32))

    return sampled
